# Initial kernel scaffold; baseline (speedup 1.0000x reference)
import jax
import jax.numpy as jnp
from jax.experimental import pallas as pl


def kernel(inputs, embeddings):
    raise NotImplementedError("write your pallas kernel here")


# speedup vs baseline: 1.0494x; 1.0494x over previous
import functools

import jax
import jax.numpy as jnp
from jax import lax
from jax.experimental import pallas as pl
from jax.experimental.pallas import tpu as pltpu
from jax.experimental.pallas import tpu_sc as plsc

D = 64
L = 16
NC = 2
NS = 16
NW = NC * NS
CHUNK = 128


def _rsqrt_newton(x):
    xi = lax.bitcast_convert_type(x, jnp.int32)
    yi = jnp.int32(0x5F3759DF) - lax.shift_right_logical(xi, 1)
    y = lax.bitcast_convert_type(yi, jnp.float32)
    xh = x * jnp.float32(0.5)
    for _ in range(3):
        y = y * (jnp.float32(1.5) - xh * y * y)
    return y


def _l2_normalize_rows(buf, ss_buf, y_buf, n_rows):
    iota16 = lax.iota(jnp.int32, L)
    splats = [jnp.full((L,), c, dtype=jnp.int32) for c in range(L)]
    y_splats = [jnp.full((L,), L + c, dtype=jnp.int32) for c in range(L)]

    def block_body(bi, carry):
        rb = bi * L
        for r in range(L):
            v = [buf[rb + r, pl.ds(L * k, L)] for k in range(D // L)]
            ss = v[0] * v[0]
            for k in range(1, D // L):
                ss = ss + v[k] * v[k]
            ss_buf[r, pl.ds(0, L)] = ss
        tot = plsc.load_gather(ss_buf, [iota16, splats[0]])
        for c in range(1, L):
            tot = tot + plsc.load_gather(ss_buf, [iota16, splats[c]])
        tot = jnp.maximum(tot, jnp.float32(1e-12))
        y_buf[pl.ds(L, L)] = _rsqrt_newton(tot)
        for r in range(L):
            yr = plsc.load_gather(y_buf, [y_splats[r]])
            for k in range(D // L):
                buf[rb + r, pl.ds(L * k, L)] = buf[rb + r, pl.ds(L * k, L)] * yr
        return carry

    lax.fori_loop(0, n_rows // L, block_body, 0)


def kernel(inputs, embeddings):
    batch, hist = inputs.shape
    n_total = batch * hist
    per_w = n_total // NW
    n_ch = per_w // CHUNK
    idx = inputs.astype(jnp.int32).reshape(NW, n_ch, CHUNK)

    mesh = plsc.VectorSubcoreMesh(core_axis_name="c", subcore_axis_name="s")

    @functools.partial(
        pl.kernel,
        out_type=jax.ShapeDtypeStruct((n_total, D), jnp.float32),
        mesh=mesh,
        scratch_types=[
            pltpu.VMEM((n_ch, CHUNK), jnp.int32),
            pltpu.VMEM((CHUNK, D), jnp.float32),
            pltpu.VMEM((L, L), jnp.float32),
            pltpu.VMEM((2 * L,), jnp.float32),
            pltpu.SemaphoreType.DMA,
        ],
        compiler_params=pltpu.CompilerParams(
            needs_layout_passes=False, use_tc_tiling_on_sc=False
        ),
    )
    def sc_kernel(table_hbm, idx_hbm, out_hbm, idx_v, buf, ss_buf, y_buf, sem):
        wid = lax.axis_index("s") * NC + lax.axis_index("c")
        pltpu.sync_copy(idx_hbm.at[wid], idx_v)
        base = wid * per_w

        def chunk_body(j, carry):
            pltpu.async_copy(table_hbm.at[idx_v.at[j]], buf, sem).wait()
            _l2_normalize_rows(buf, ss_buf, y_buf, CHUNK)
            pltpu.sync_copy(buf, out_hbm.at[pl.ds(base + j * CHUNK, CHUNK)])
            return carry

        lax.fori_loop(0, n_ch, chunk_body, 0)

    out = sc_kernel(embeddings, idx)
    return out.reshape(batch, hist, D)


# speedup vs baseline: 1.0508x; 1.0013x over previous
import functools

import jax
import jax.numpy as jnp
from jax import lax
from jax.experimental import pallas as pl
from jax.experimental.pallas import tpu as pltpu
from jax.experimental.pallas import tpu_sc as plsc

D = 64
L = 16
NC = 2
NS = 16
NW = NC * NS
CHUNK = 128


def _rsqrt_newton(x):
    xi = lax.bitcast_convert_type(x, jnp.int32)
    yi = jnp.int32(0x5F3759DF) - lax.shift_right_logical(xi, 1)
    y = lax.bitcast_convert_type(yi, jnp.float32)
    xh = x * jnp.float32(0.5)
    for _ in range(3):
        y = y * (jnp.float32(1.5) - xh * y * y)
    return y


def _l2_normalize_rows(buf, ss_buf, y_buf, n_rows):
    iota16 = lax.iota(jnp.int32, L)
    splats = [jnp.full((L,), c, dtype=jnp.int32) for c in range(L)]
    y_splats = [jnp.full((L,), L + c, dtype=jnp.int32) for c in range(L)]

    def block_body(bi, carry):
        rb = bi * L
        for r in range(L):
            v = [buf[rb + r, pl.ds(L * k, L)] for k in range(D // L)]
            ss = v[0] * v[0]
            for k in range(1, D // L):
                ss = ss + v[k] * v[k]
            ss_buf[r, pl.ds(0, L)] = ss
        tot = plsc.load_gather(ss_buf, [iota16, splats[0]])
        for c in range(1, L):
            tot = tot + plsc.load_gather(ss_buf, [iota16, splats[c]])
        tot = jnp.maximum(tot, jnp.float32(1e-12))
        y_buf[pl.ds(L, L)] = _rsqrt_newton(tot)
        for r in range(L):
            yr = plsc.load_gather(y_buf, [y_splats[r]])
            for k in range(D // L):
                buf[rb + r, pl.ds(L * k, L)] = buf[rb + r, pl.ds(L * k, L)] * yr
        return carry

    lax.fori_loop(0, n_rows // L, block_body, 0)


def kernel(inputs, embeddings):
    batch, hist = inputs.shape
    vocab = embeddings.shape[0]
    n_total = batch * hist
    per_w = n_total // NW
    n_ch = per_w // CHUNK
    idx = inputs.astype(jnp.int32).reshape(NW, n_ch, CHUNK)

    mesh = plsc.VectorSubcoreMesh(core_axis_name="c", subcore_axis_name="s")

    @functools.partial(
        pl.kernel,
        out_type=jax.ShapeDtypeStruct((n_total, D), jnp.float32),
        mesh=mesh,
        scratch_types=[
            pltpu.VMEM((n_ch, CHUNK), jnp.int32),
            pltpu.VMEM((CHUNK, D), jnp.float32),
            pltpu.VMEM((L, L), jnp.float32),
            pltpu.VMEM((2 * L,), jnp.float32),
            pltpu.SemaphoreType.DMA,
        ],
        compiler_params=pltpu.CompilerParams(
            needs_layout_passes=False, use_tc_tiling_on_sc=False
        ),
    )
    def sc_kernel(table_hbm, idx_hbm, out_hbm, idx_v, buf, ss_buf, y_buf, sem):
        wid = lax.axis_index("s") * NC + lax.axis_index("c")
        pltpu.sync_copy(idx_hbm.at[wid], idx_v)
        base = wid * per_w

        def chunk_body(j, carry):
            pltpu.async_copy(table_hbm.at[idx_v.at[j]], buf, sem).wait()
            _l2_normalize_rows(buf, ss_buf, y_buf, CHUNK)
            pltpu.sync_copy(buf, out_hbm.at[pl.ds(base + j * CHUNK, CHUNK)])
            return carry

        lax.fori_loop(0, n_ch, chunk_body, 0)

    emb_lin = lax.optimization_barrier(embeddings.reshape(-1))
    out = sc_kernel(emb_lin.reshape(vocab, D), idx)
    out_lin = lax.optimization_barrier(out.reshape(-1))
    return out_lin.reshape(batch, hist, D)
